# Initial kernel scaffold; baseline (speedup 1.0000x reference)
import jax
import jax.numpy as jnp
from jax.experimental import pallas as pl


def kernel(alpha, beta, edge_index0, edge_index1, edge_index2, score0, score1):
    raise NotImplementedError("write your pallas kernel here")


# speedup vs baseline: 3.6483x; 3.6483x over previous
import functools

import jax
import jax.numpy as jnp
from jax import lax
from jax.experimental import pallas as pl
from jax.experimental.pallas import tpu as pltpu
from jax.experimental.pallas import tpu_sc as plsc

B = 4096
E = 131072
NPLANE = 5
PLANE = B * B
TOTAL = NPLANE * PLANE

NC = 2
NS = 16
NW = NC * NS
CHUNK = E // NW
NKEY = 2 * CHUNK
KROWS = NKEY // 128
LANES = 16


def _scatter_body(e0, e1, e2, e3, e4, planes, rbuf, cbuf, kbuf, ones, sem):
    wid = lax.axis_index("s") * NC + lax.axis_index("c")
    base = wid * CHUNK

    for i in range(128 // LANES):
        ones[pl.ds(i * LANES, LANES)] = jnp.full((LANES,), 1.0, jnp.float32)

    for plane_idx, e in enumerate((e0, e1, e2, e3, e4)):
        off = plane_idx * PLANE
        pltpu.sync_copy(e.at[0, pl.ds(base, CHUNK)], cbuf)
        pltpu.sync_copy(e.at[1, pl.ds(base, CHUNK)], rbuf)

        def compute(t, _):
            r = rbuf[pl.ds(t * LANES, LANES)]
            c = cbuf[pl.ds(t * LANES, LANES)]
            k1 = r * B + c + off
            k2 = c * B + r + off
            row = t // 8
            col = (t % 8) * LANES
            kbuf[row, pl.ds(col, LANES)] = k1
            kbuf[KROWS // 2 + row, pl.ds(col, LANES)] = k2
            return 0

        lax.fori_loop(0, CHUNK // LANES, compute, 0, unroll=2)

        def issue(j, _):
            pltpu.async_copy(ones, planes.at[kbuf.at[j]], sem)
            return 0

        lax.fori_loop(0, KROWS, issue, 0)

        def drain(j, _):
            pltpu.make_async_copy(ones, planes.at[kbuf.at[j]], sem).wait()
            return 0

        lax.fori_loop(0, KROWS, drain, 0)


@functools.cache
def _make_scatter():
    return pl.kernel(
        _scatter_body,
        out_type=(),
        mesh=plsc.VectorSubcoreMesh(
            core_axis_name="c",
            subcore_axis_name="s",
            num_cores=NC,
            num_subcores=NS,
        ),
        scratch_types=[
            pltpu.VMEM((CHUNK,), jnp.int32),
            pltpu.VMEM((CHUNK,), jnp.int32),
            pltpu.VMEM((KROWS, 128), jnp.int32),
            pltpu.VMEM((128,), jnp.float32),
            pltpu.SemaphoreType.DMA,
        ],
    )


RB = 64
NBLK = B // RB


def _reduce_body(p_ref, o_ref):
    pid = pl.program_id(0)
    a0 = p_ref[0]
    a1 = p_ref[1]
    a2 = p_ref[2]
    s0 = p_ref[3]
    s1 = p_ref[4]
    not2 = 1.0 - a2
    d0 = a0 * (1.0 - a1) * not2
    d1 = a1 * (1.0 - a0) * not2
    bsum = jnp.sum(jnp.abs(d0 - s0) + jnp.abs(d1 - s1))

    @pl.when(pid == 0)
    def _():
        o_ref[0, 0] = 0.0

    o_ref[0, 0] += bsum

    @pl.when(pid == NBLK - 1)
    def _():
        o_ref[0, 0] = o_ref[0, 0] * (1.0 / (2 * B * B))


_reduce = pl.pallas_call(
    _reduce_body,
    grid=(NBLK,),
    in_specs=[pl.BlockSpec((NPLANE, RB, B), lambda i: (0, i, 0))],
    out_specs=pl.BlockSpec(memory_space=pltpu.SMEM),
    out_shape=jax.ShapeDtypeStruct((1, 1), jnp.float32),
)


def kernel(alpha, beta, edge_index0, edge_index1, edge_index2, score0, score1):
    del alpha, beta
    edges = [
        e.astype(jnp.int32)
        for e in (edge_index0, edge_index1, edge_index2, score0, score1)
    ]
    planes_ref = jax.new_ref(jnp.zeros((TOTAL,), jnp.float32))
    _make_scatter()(*edges, planes_ref)
    planes = planes_ref[...].reshape(NPLANE, B, B)
    return _reduce(planes)[0, 0]
